# baseline (speedup 1.0000x reference)
import functools

import jax
import jax.numpy as jnp
from jax import lax
from jax.experimental import pallas as pl
from jax.experimental.pallas import tpu as pltpu
from jax.experimental.pallas import tpu_sc as plsc

_N = 50000
_E = 800000
_DIN = 128
_DH = 64
_HF = 32
_NG = 512
_NS = 16
_CW = 125
_RE = _E // _CW
_RET = _RE // _NS
_STN = _N // _NS
_RN = _N // _CW
_RNT = _RN // _NS
_K = 4
_KP = 8


def _sc_mesh():
    return plsc.VectorSubcoreMesh(core_axis_name="c", subcore_axis_name="s")


def _prep_body(dst_rs, dinv_out, degacc, ones_t, idx, stripe_t, ssem):
    c = lax.axis_index("c")
    s = lax.axis_index("s")

    def fill_ones(i, carry):
        ones_t[i, :] = jnp.full((16,), 1.0, jnp.float32)
        return carry
    lax.fori_loop(0, _CW, fill_ones, 0)

    @pl.when(c == 0)
    def _():
        def init(i, carry):
            pltpu.sync_copy(ones_t, degacc.at[pl.ds(s * _STN + i * _CW, _CW)])
            return carry
        lax.fori_loop(0, _STN // _CW, init, 0)

    plsc.subcore_barrier()

    @pl.when(c == 0)
    def _():
        def chunk(cc, carry):
            r0 = s * _RET + cc * _KP
            pltpu.sync_copy(dst_rs.at[pl.ds(r0, _KP)], idx)
            for j in range(_KP):
                pltpu.async_copy(ones_t, degacc.at[idx.at[j]], ssem,
                                 add=True)
            for j in range(_KP):
                pltpu.make_async_copy(ones_t, degacc.at[idx.at[j]],
                                      ssem).wait()
            return carry
        lax.fori_loop(0, _RET // _KP, chunk, 0)

    plsc.subcore_barrier()

    @pl.when(c == 0)
    def _():
        pltpu.sync_copy(degacc.at[pl.ds(s * _STN, _STN)], stripe_t)

        def newton(i, carry):
            d = stripe_t[i, :]
            bits = plsc.bitcast(d, jnp.int32)
            y = plsc.bitcast(jnp.int32(0x5F3759DF) - (bits >> 1), jnp.float32)
            hd = d * 0.5
            y = y * (1.5 - hd * y * y)
            y = y * (1.5 - hd * y * y)
            y = y * (1.5 - hd * y * y)
            stripe_t[i, :] = y
            return carry
        lax.fori_loop(0, _STN, newton, 0)
        pltpu.sync_copy(stripe_t, dinv_out.at[pl.ds(s * _STN, _STN)])


@jax.jit
def _prep(dst_rs):
    f = pl.kernel(
        _prep_body,
        out_type=jax.ShapeDtypeStruct((_N, 16), jnp.float32),
        mesh=_sc_mesh(),
        compiler_params=pltpu.CompilerParams(use_tc_tiling_on_sc=False, needs_layout_passes=False),
        scratch_types=[
            pltpu.VMEM_SHARED((_N, 16), jnp.float32),
            pltpu.VMEM((_CW, 16), jnp.float32),
            pltpu.VMEM((_KP, _CW), jnp.int32),
            pltpu.VMEM((_STN, 16), jnp.float32),
            pltpu.SemaphoreType.DMA,
        ],
    )
    return f(dst_rs)


def _agg_body(g_lo, g_hi, src_rs, dst_rs, zeros, out_lo, out_hi,
              acc, idx_g, idx_s, rows, gsem, ssem, isem):
    c = lax.axis_index("c")
    s = lax.axis_index("s")
    pltpu.sync_copy(zeros.at[pl.ds(s * _STN, _STN)],
                    acc.at[pl.ds(s * _STN, _STN)])
    plsc.subcore_barrier()

    def edge_loop(table):
        base = s * _RET
        nchunk = _RET // _K
        pltpu.async_copy(src_rs.at[pl.ds(base, _K)], idx_g.at[0], isem)
        pltpu.async_copy(dst_rs.at[pl.ds(base, _K)], idx_s.at[0], isem)

        def chunk(cc, carry):
            cur = lax.rem(cc, 2)
            nxt = lax.rem(cc + 1, 2)
            pltpu.make_async_copy(src_rs.at[pl.ds(base, _K)],
                                  idx_g.at[cur], isem).wait()
            pltpu.make_async_copy(dst_rs.at[pl.ds(base, _K)],
                                  idx_s.at[cur], isem).wait()

            @pl.when(cc + 1 < nchunk)
            def _():
                r1 = base + (cc + 1) * _K
                pltpu.async_copy(src_rs.at[pl.ds(r1, _K)], idx_g.at[nxt],
                                 isem)
                pltpu.async_copy(dst_rs.at[pl.ds(r1, _K)], idx_s.at[nxt],
                                 isem)
            for j in range(_K):
                pltpu.async_copy(table.at[idx_g.at[cur, j]], rows.at[j], gsem)
            for j in range(_K):
                pltpu.make_async_copy(table.at[idx_g.at[cur, j]], rows.at[j],
                                      gsem).wait()
                pltpu.async_copy(rows.at[j], acc.at[idx_s.at[cur, j]], ssem,
                                 add=True)
            for j in range(_K):
                pltpu.make_async_copy(rows.at[j], acc.at[idx_s.at[cur, j]],
                                      ssem).wait()
            return carry
        lax.fori_loop(0, nchunk, chunk, 0)

    @pl.when(c == 0)
    def _():
        edge_loop(g_lo)

    @pl.when(c == 1)
    def _():
        edge_loop(g_hi)

    plsc.subcore_barrier()

    @pl.when(c == 0)
    def _():
        pltpu.sync_copy(acc.at[pl.ds(s * _STN, _STN)],
                        out_lo.at[pl.ds(s * _STN, _STN)])

    @pl.when(c == 1)
    def _():
        pltpu.sync_copy(acc.at[pl.ds(s * _STN, _STN)],
                        out_hi.at[pl.ds(s * _STN, _STN)])


@jax.jit
def _agg(g_lo, g_hi, src_rs, dst_rs, zeros):
    f = pl.kernel(
        _agg_body,
        out_type=(jax.ShapeDtypeStruct((_N, _HF), jnp.float32),
                  jax.ShapeDtypeStruct((_N, _HF), jnp.float32)),
        mesh=_sc_mesh(),
        compiler_params=pltpu.CompilerParams(use_tc_tiling_on_sc=False, needs_layout_passes=False),
        scratch_types=[
            pltpu.VMEM_SHARED((_N, _HF), jnp.float32),
            pltpu.VMEM((2, _K, _CW), jnp.int32),
            pltpu.VMEM((2, _K, _CW), jnp.int32),
            pltpu.VMEM((_K, _CW, _HF), jnp.float32),
            pltpu.SemaphoreType.DMA,
            pltpu.SemaphoreType.DMA,
            pltpu.SemaphoreType.DMA,
        ],
    )
    return f(g_lo, g_hi, src_rs, dst_rs, zeros)


_BLK = 2000


def _tc1_body(x_ref, w_ref, dinv_ref, lo_ref, hi_ref):
    g = jnp.dot(x_ref[...], w_ref[...], preferred_element_type=jnp.float32)
    g = g * dinv_ref[:, :1]
    lo_ref[...] = g[:, :_HF]
    hi_ref[...] = g[:, _HF:]


@jax.jit
def _tc1(x, W1, dinv_w):
    return pl.pallas_call(
        _tc1_body,
        grid=(_N // _BLK,),
        in_specs=[
            pl.BlockSpec((_BLK, _DIN), lambda i: (i, 0)),
            pl.BlockSpec((_DIN, _DH), lambda i: (0, 0)),
            pl.BlockSpec((_BLK, 16), lambda i: (i, 0)),
        ],
        out_specs=[pl.BlockSpec((_BLK, _HF), lambda i: (i, 0)),
                   pl.BlockSpec((_BLK, _HF), lambda i: (i, 0))],
        out_shape=[jax.ShapeDtypeStruct((_N, _HF), jnp.float32),
                   jax.ShapeDtypeStruct((_N, _HF), jnp.float32)],
    )(x, W1, dinv_w)


def _tcmid_body(slo_ref, shi_ref, glo_ref, ghi_ref, dinv_ref, b_ref, w_ref,
                lo_ref, hi_ref):
    dinv = dinv_ref[:, :1]
    sf = jnp.concatenate([slo_ref[...], shi_ref[...]], axis=1)
    gf = jnp.concatenate([glo_ref[...], ghi_ref[...]], axis=1)
    xn = jnp.maximum(dinv * (sf + gf) + b_ref[...], 0.0)
    g2 = jnp.dot(xn, w_ref[...], preferred_element_type=jnp.float32) * dinv
    lo_ref[...] = g2[:, :_HF]
    hi_ref[...] = g2[:, _HF:]


@jax.jit
def _tcmid(s_lo, s_hi, g_lo, g_hi, dinv_w, b_prev, W):
    blk = lambda i: (i, 0)
    return pl.pallas_call(
        _tcmid_body,
        grid=(_N // _BLK,),
        in_specs=[
            pl.BlockSpec((_BLK, _HF), blk),
            pl.BlockSpec((_BLK, _HF), blk),
            pl.BlockSpec((_BLK, _HF), blk),
            pl.BlockSpec((_BLK, _HF), blk),
            pl.BlockSpec((_BLK, 16), blk),
            pl.BlockSpec((1, _DH), lambda i: (0, 0)),
            pl.BlockSpec((_DH, _DH), lambda i: (0, 0)),
        ],
        out_specs=[pl.BlockSpec((_BLK, _HF), blk),
                   pl.BlockSpec((_BLK, _HF), blk)],
        out_shape=[jax.ShapeDtypeStruct((_N, _HF), jnp.float32),
                   jax.ShapeDtypeStruct((_N, _HF), jnp.float32)],
    )(s_lo, s_hi, g_lo, g_hi, dinv_w, b_prev, W)


def _poolmm_body(slo_ref, shi_ref, glo_ref, ghi_ref, dinv_ref, batch_ref,
                 b3_ref, wl_ref, bl_ref, out_ref, sum_ref, cnt_ref):
    i = pl.program_id(0)
    dinv = dinv_ref[:, :1]
    sf = jnp.concatenate([slo_ref[...], shi_ref[...]], axis=1)
    gf = jnp.concatenate([glo_ref[...], ghi_ref[...]], axis=1)
    h = dinv * (sf + gf)
    ids = batch_ref[:, :1]
    gidx = lax.broadcasted_iota(jnp.int32, (_BLK, _NG), 1)
    oh = (ids == gidx).astype(jnp.float32)
    psum = lax.dot_general(oh, h, (((0,), (0,)), ((), ())),
                           preferred_element_type=jnp.float32)
    pcnt = jnp.sum(oh, axis=0)[None, :]

    @pl.when(i == 0)
    def _():
        sum_ref[...] = jnp.zeros_like(sum_ref)
        cnt_ref[...] = jnp.zeros_like(cnt_ref)
    sum_ref[...] += psum
    cnt_ref[...] += pcnt

    @pl.when(i == _N // _BLK - 1)
    def _():
        t = sum_ref[...]
        cnt = jnp.reshape(cnt_ref[0], (_NG, 1))
        pooled = (t + cnt * b3_ref[...]) / jnp.maximum(cnt, 1.0)
        out_ref[...] = (jnp.dot(pooled, wl_ref[...],
                                preferred_element_type=jnp.float32)
                        + bl_ref[...])


@jax.jit
def _poolmm(s_lo, s_hi, g_lo, g_hi, dinv_w, batch_col, b3, Wl, bl):
    return pl.pallas_call(
        _poolmm_body,
        grid=(_N // _BLK,),
        in_specs=[
            pl.BlockSpec((_BLK, _HF), lambda i: (i, 0)),
            pl.BlockSpec((_BLK, _HF), lambda i: (i, 0)),
            pl.BlockSpec((_BLK, _HF), lambda i: (i, 0)),
            pl.BlockSpec((_BLK, _HF), lambda i: (i, 0)),
            pl.BlockSpec((_BLK, 16), lambda i: (i, 0)),
            pl.BlockSpec((_BLK, 1), lambda i: (i, 0)),
            pl.BlockSpec((1, _DH), lambda i: (0, 0)),
            pl.BlockSpec((_DH, 8), lambda i: (0, 0)),
            pl.BlockSpec((1, 8), lambda i: (0, 0)),
        ],
        out_specs=pl.BlockSpec((_NG, 8), lambda i: (0, 0)),
        out_shape=jax.ShapeDtypeStruct((_NG, 8), jnp.float32),
        scratch_shapes=[
            pltpu.VMEM((_NG, _DH), jnp.float32),
            pltpu.VMEM((1, _NG), jnp.float32),
        ],
    )(s_lo, s_hi, g_lo, g_hi, dinv_w, batch_col, b3, Wl, bl)


def kernel(x, edge_index, batch, W1, b1, W2, b2, W3, b3, Wl, bl):
    src = edge_index[0].astype(jnp.int32)
    dst = edge_index[1].astype(jnp.int32)
    src_rs = src.reshape(_RE, _CW)
    dst_rs = dst.reshape(_RE, _CW)
    batch_col = batch.astype(jnp.int32).reshape(_N, 1)
    zeros = jnp.zeros((_N, _HF), jnp.float32)

    dinv_w = _prep(dst_rs)
    g1_lo, g1_hi = _tc1(x, W1, dinv_w)
    s1_lo, s1_hi = _agg(g1_lo, g1_hi, src_rs, dst_rs, zeros)
    g2_lo, g2_hi = _tcmid(s1_lo, s1_hi, g1_lo, g1_hi, dinv_w,
                          b1.reshape(1, _DH), W2)
    s2_lo, s2_hi = _agg(g2_lo, g2_hi, src_rs, dst_rs, zeros)
    g3_lo, g3_hi = _tcmid(s2_lo, s2_hi, g2_lo, g2_hi, dinv_w,
                          b2.reshape(1, _DH), W3)
    s3_lo, s3_hi = _agg(g3_lo, g3_hi, src_rs, dst_rs, zeros)
    return _poolmm(s3_lo, s3_hi, g3_lo, g3_hi, dinv_w, batch_col,
                   b3.reshape(1, _DH), Wl, bl.reshape(1, 8))


# speedup vs baseline: 1.0525x; 1.0525x over previous
import functools

import jax
import jax.numpy as jnp
from jax import lax
from jax.experimental import pallas as pl
from jax.experimental.pallas import tpu as pltpu
from jax.experimental.pallas import tpu_sc as plsc

_N = 50000
_E = 800000
_DIN = 128
_DH = 64
_HF = 32
_NG = 512
_NS = 16
_CW = 125
_RE = _E // _CW
_RET = _RE // _NS
_STN = _N // _NS
_RN = _N // _CW
_RNT = _RN // _NS
_K = 5
_KP = 8


def _sc_mesh():
    return plsc.VectorSubcoreMesh(core_axis_name="c", subcore_axis_name="s")


def _prep_body(dst_rs, dinv_out, degacc, ones_t, idx, stripe_t, ssem):
    c = lax.axis_index("c")
    s = lax.axis_index("s")

    def fill_ones(i, carry):
        ones_t[i, :] = jnp.full((16,), 1.0, jnp.float32)
        return carry
    lax.fori_loop(0, _CW, fill_ones, 0)

    @pl.when(c == 0)
    def _():
        def init(i, carry):
            pltpu.sync_copy(ones_t, degacc.at[pl.ds(s * _STN + i * _CW, _CW)])
            return carry
        lax.fori_loop(0, _STN // _CW, init, 0)

    plsc.subcore_barrier()

    @pl.when(c == 0)
    def _():
        def chunk(cc, carry):
            r0 = s * _RET + cc * _KP
            pltpu.sync_copy(dst_rs.at[pl.ds(r0, _KP)], idx)
            for j in range(_KP):
                pltpu.async_copy(ones_t, degacc.at[idx.at[j]], ssem,
                                 add=True)
            for j in range(_KP):
                pltpu.make_async_copy(ones_t, degacc.at[idx.at[j]],
                                      ssem).wait()
            return carry
        lax.fori_loop(0, _RET // _KP, chunk, 0)

    plsc.subcore_barrier()

    @pl.when(c == 0)
    def _():
        pltpu.sync_copy(degacc.at[pl.ds(s * _STN, _STN)], stripe_t)

        def newton(i, carry):
            d = stripe_t[i, :]
            bits = plsc.bitcast(d, jnp.int32)
            y = plsc.bitcast(jnp.int32(0x5F3759DF) - (bits >> 1), jnp.float32)
            hd = d * 0.5
            y = y * (1.5 - hd * y * y)
            y = y * (1.5 - hd * y * y)
            y = y * (1.5 - hd * y * y)
            stripe_t[i, :] = y
            return carry
        lax.fori_loop(0, _STN, newton, 0)
        pltpu.sync_copy(stripe_t, dinv_out.at[pl.ds(s * _STN, _STN)])


@jax.jit
def _prep(dst_rs):
    f = pl.kernel(
        _prep_body,
        out_type=jax.ShapeDtypeStruct((_N, 16), jnp.float32),
        mesh=_sc_mesh(),
        compiler_params=pltpu.CompilerParams(use_tc_tiling_on_sc=False, needs_layout_passes=False),
        scratch_types=[
            pltpu.VMEM_SHARED((_N, 16), jnp.float32),
            pltpu.VMEM((_CW, 16), jnp.float32),
            pltpu.VMEM((_KP, _CW), jnp.int32),
            pltpu.VMEM((_STN, 16), jnp.float32),
            pltpu.SemaphoreType.DMA,
        ],
    )
    return f(dst_rs)


def _agg_body(g_lo, g_hi, src_rs, dst_rs, zeros, out_lo, out_hi,
              acc, idx_g, idx_s, rows, gsem, ssem, isem):
    c = lax.axis_index("c")
    s = lax.axis_index("s")
    pltpu.sync_copy(zeros.at[pl.ds(s * _STN, _STN)],
                    acc.at[pl.ds(s * _STN, _STN)])
    plsc.subcore_barrier()

    def edge_loop(table):
        base = s * _RET
        nchunk = _RET // _K
        pltpu.async_copy(src_rs.at[pl.ds(base, _K)], idx_g.at[0], isem)
        pltpu.async_copy(dst_rs.at[pl.ds(base, _K)], idx_s.at[0], isem)

        def chunk(cc, carry):
            cur = lax.rem(cc, 2)
            nxt = lax.rem(cc + 1, 2)
            pltpu.make_async_copy(src_rs.at[pl.ds(base, _K)],
                                  idx_g.at[cur], isem).wait()
            pltpu.make_async_copy(dst_rs.at[pl.ds(base, _K)],
                                  idx_s.at[cur], isem).wait()

            @pl.when(cc + 1 < nchunk)
            def _():
                r1 = base + (cc + 1) * _K
                pltpu.async_copy(src_rs.at[pl.ds(r1, _K)], idx_g.at[nxt],
                                 isem)
                pltpu.async_copy(dst_rs.at[pl.ds(r1, _K)], idx_s.at[nxt],
                                 isem)
            for j in range(_K):
                pltpu.async_copy(table.at[idx_g.at[cur, j]], rows.at[j], gsem)
            for j in range(_K):
                pltpu.make_async_copy(table.at[idx_g.at[cur, j]], rows.at[j],
                                      gsem).wait()
                pltpu.async_copy(rows.at[j], acc.at[idx_s.at[cur, j]], ssem,
                                 add=True)
            for j in range(_K):
                pltpu.make_async_copy(rows.at[j], acc.at[idx_s.at[cur, j]],
                                      ssem).wait()
            return carry
        lax.fori_loop(0, nchunk, chunk, 0)

    @pl.when(c == 0)
    def _():
        edge_loop(g_lo)

    @pl.when(c == 1)
    def _():
        edge_loop(g_hi)

    plsc.subcore_barrier()

    @pl.when(c == 0)
    def _():
        pltpu.sync_copy(acc.at[pl.ds(s * _STN, _STN)],
                        out_lo.at[pl.ds(s * _STN, _STN)])

    @pl.when(c == 1)
    def _():
        pltpu.sync_copy(acc.at[pl.ds(s * _STN, _STN)],
                        out_hi.at[pl.ds(s * _STN, _STN)])


@jax.jit
def _agg(g_lo, g_hi, src_rs, dst_rs, zeros):
    f = pl.kernel(
        _agg_body,
        out_type=(jax.ShapeDtypeStruct((_N, _HF), jnp.float32),
                  jax.ShapeDtypeStruct((_N, _HF), jnp.float32)),
        mesh=_sc_mesh(),
        compiler_params=pltpu.CompilerParams(use_tc_tiling_on_sc=False, needs_layout_passes=False),
        scratch_types=[
            pltpu.VMEM_SHARED((_N, _HF), jnp.float32),
            pltpu.VMEM((2, _K, _CW), jnp.int32),
            pltpu.VMEM((2, _K, _CW), jnp.int32),
            pltpu.VMEM((_K, _CW, _HF), jnp.float32),
            pltpu.SemaphoreType.DMA,
            pltpu.SemaphoreType.DMA,
            pltpu.SemaphoreType.DMA,
        ],
    )
    return f(g_lo, g_hi, src_rs, dst_rs, zeros)


_BLK = 5000


def _tc1_body(x_ref, w_ref, dinv_ref, lo_ref, hi_ref):
    g = jnp.dot(x_ref[...], w_ref[...], preferred_element_type=jnp.float32)
    g = g * dinv_ref[:, :1]
    lo_ref[...] = g[:, :_HF]
    hi_ref[...] = g[:, _HF:]


@jax.jit
def _tc1(x, W1, dinv_w):
    return pl.pallas_call(
        _tc1_body,
        grid=(_N // _BLK,),
        in_specs=[
            pl.BlockSpec((_BLK, _DIN), lambda i: (i, 0)),
            pl.BlockSpec((_DIN, _DH), lambda i: (0, 0)),
            pl.BlockSpec((_BLK, 16), lambda i: (i, 0)),
        ],
        out_specs=[pl.BlockSpec((_BLK, _HF), lambda i: (i, 0)),
                   pl.BlockSpec((_BLK, _HF), lambda i: (i, 0))],
        out_shape=[jax.ShapeDtypeStruct((_N, _HF), jnp.float32),
                   jax.ShapeDtypeStruct((_N, _HF), jnp.float32)],
    )(x, W1, dinv_w)


def _tcmid_body(slo_ref, shi_ref, glo_ref, ghi_ref, dinv_ref, b_ref, w_ref,
                lo_ref, hi_ref):
    dinv = dinv_ref[:, :1]
    sf = jnp.concatenate([slo_ref[...], shi_ref[...]], axis=1)
    gf = jnp.concatenate([glo_ref[...], ghi_ref[...]], axis=1)
    xn = jnp.maximum(dinv * (sf + gf) + b_ref[...], 0.0)
    g2 = jnp.dot(xn, w_ref[...], preferred_element_type=jnp.float32) * dinv
    lo_ref[...] = g2[:, :_HF]
    hi_ref[...] = g2[:, _HF:]


@jax.jit
def _tcmid(s_lo, s_hi, g_lo, g_hi, dinv_w, b_prev, W):
    blk = lambda i: (i, 0)
    return pl.pallas_call(
        _tcmid_body,
        grid=(_N // _BLK,),
        in_specs=[
            pl.BlockSpec((_BLK, _HF), blk),
            pl.BlockSpec((_BLK, _HF), blk),
            pl.BlockSpec((_BLK, _HF), blk),
            pl.BlockSpec((_BLK, _HF), blk),
            pl.BlockSpec((_BLK, 16), blk),
            pl.BlockSpec((1, _DH), lambda i: (0, 0)),
            pl.BlockSpec((_DH, _DH), lambda i: (0, 0)),
        ],
        out_specs=[pl.BlockSpec((_BLK, _HF), blk),
                   pl.BlockSpec((_BLK, _HF), blk)],
        out_shape=[jax.ShapeDtypeStruct((_N, _HF), jnp.float32),
                   jax.ShapeDtypeStruct((_N, _HF), jnp.float32)],
    )(s_lo, s_hi, g_lo, g_hi, dinv_w, b_prev, W)


def _poolmm_body(slo_ref, shi_ref, glo_ref, ghi_ref, dinv_ref, batch_ref,
                 b3_ref, wl_ref, bl_ref, out_ref, sum_ref, cnt_ref):
    i = pl.program_id(0)
    dinv = dinv_ref[:, :1]
    sf = jnp.concatenate([slo_ref[...], shi_ref[...]], axis=1)
    gf = jnp.concatenate([glo_ref[...], ghi_ref[...]], axis=1)
    h = dinv * (sf + gf)
    ids = batch_ref[:, :1]
    gidx = lax.broadcasted_iota(jnp.int32, (_BLK, _NG), 1)
    oh = (ids == gidx).astype(jnp.float32)
    psum = lax.dot_general(oh, h, (((0,), (0,)), ((), ())),
                           preferred_element_type=jnp.float32)
    pcnt = jnp.sum(oh, axis=0)[None, :]

    @pl.when(i == 0)
    def _():
        sum_ref[...] = jnp.zeros_like(sum_ref)
        cnt_ref[...] = jnp.zeros_like(cnt_ref)
    sum_ref[...] += psum
    cnt_ref[...] += pcnt

    @pl.when(i == _N // _BLK - 1)
    def _():
        t = sum_ref[...]
        cnt = jnp.reshape(cnt_ref[0], (_NG, 1))
        pooled = (t + cnt * b3_ref[...]) / jnp.maximum(cnt, 1.0)
        out_ref[...] = (jnp.dot(pooled, wl_ref[...],
                                preferred_element_type=jnp.float32)
                        + bl_ref[...])


@jax.jit
def _poolmm(s_lo, s_hi, g_lo, g_hi, dinv_w, batch_col, b3, Wl, bl):
    return pl.pallas_call(
        _poolmm_body,
        grid=(_N // _BLK,),
        in_specs=[
            pl.BlockSpec((_BLK, _HF), lambda i: (i, 0)),
            pl.BlockSpec((_BLK, _HF), lambda i: (i, 0)),
            pl.BlockSpec((_BLK, _HF), lambda i: (i, 0)),
            pl.BlockSpec((_BLK, _HF), lambda i: (i, 0)),
            pl.BlockSpec((_BLK, 16), lambda i: (i, 0)),
            pl.BlockSpec((_BLK, 1), lambda i: (i, 0)),
            pl.BlockSpec((1, _DH), lambda i: (0, 0)),
            pl.BlockSpec((_DH, 8), lambda i: (0, 0)),
            pl.BlockSpec((1, 8), lambda i: (0, 0)),
        ],
        out_specs=pl.BlockSpec((_NG, 8), lambda i: (0, 0)),
        out_shape=jax.ShapeDtypeStruct((_NG, 8), jnp.float32),
        scratch_shapes=[
            pltpu.VMEM((_NG, _DH), jnp.float32),
            pltpu.VMEM((1, _NG), jnp.float32),
        ],
    )(s_lo, s_hi, g_lo, g_hi, dinv_w, batch_col, b3, Wl, bl)


def kernel(x, edge_index, batch, W1, b1, W2, b2, W3, b3, Wl, bl):
    src = edge_index[0].astype(jnp.int32)
    dst = edge_index[1].astype(jnp.int32)
    src_rs = src.reshape(_RE, _CW)
    dst_rs = dst.reshape(_RE, _CW)
    batch_col = batch.astype(jnp.int32).reshape(_N, 1)
    zeros = jnp.zeros((_N, _HF), jnp.float32)

    dinv_w = _prep(dst_rs)
    g1_lo, g1_hi = _tc1(x, W1, dinv_w)
    s1_lo, s1_hi = _agg(g1_lo, g1_hi, src_rs, dst_rs, zeros)
    g2_lo, g2_hi = _tcmid(s1_lo, s1_hi, g1_lo, g1_hi, dinv_w,
                          b1.reshape(1, _DH), W2)
    s2_lo, s2_hi = _agg(g2_lo, g2_hi, src_rs, dst_rs, zeros)
    g3_lo, g3_hi = _tcmid(s2_lo, s2_hi, g2_lo, g2_hi, dinv_w,
                          b2.reshape(1, _DH), W3)
    s3_lo, s3_hi = _agg(g3_lo, g3_hi, src_rs, dst_rs, zeros)
    return _poolmm(s3_lo, s3_hi, g3_lo, g3_hi, dinv_w, batch_col,
                   b3.reshape(1, _DH), Wl, bl.reshape(1, 8))


# speedup vs baseline: 1.0943x; 1.0397x over previous
import functools

import jax
import jax.numpy as jnp
from jax import lax
from jax.experimental import pallas as pl
from jax.experimental.pallas import tpu as pltpu
from jax.experimental.pallas import tpu_sc as plsc

_N = 50000
_E = 800000
_DIN = 128
_DH = 64
_HF = 32
_NG = 512
_NS = 16
_CW = 125
_RE = _E // _CW
_RET = _RE // _NS
_STN = _N // _NS
_RN = _N // _CW
_RNT = _RN // _NS
_K = 5
_KP = 8


def _sc_mesh():
    return plsc.VectorSubcoreMesh(core_axis_name="c", subcore_axis_name="s")


def _prep_body(dst_rs, dinv_out, degacc, ones_t, idx, stripe_t, ssem):
    c = lax.axis_index("c")
    s = lax.axis_index("s")

    def fill_ones(i, carry):
        ones_t[i, :] = jnp.full((16,), 1.0, jnp.float32)
        return carry
    lax.fori_loop(0, _CW, fill_ones, 0)

    @pl.when(c == 0)
    def _():
        def init(i, carry):
            pltpu.sync_copy(ones_t, degacc.at[pl.ds(s * _STN + i * _CW, _CW)])
            return carry
        lax.fori_loop(0, _STN // _CW, init, 0)

    plsc.subcore_barrier()

    @pl.when(c == 0)
    def _():
        def chunk(cc, carry):
            r0 = s * _RET + cc * _KP
            pltpu.sync_copy(dst_rs.at[pl.ds(r0, _KP)], idx)
            for j in range(_KP):
                pltpu.async_copy(ones_t, degacc.at[idx.at[j]], ssem,
                                 add=True)
            for j in range(_KP):
                pltpu.make_async_copy(ones_t, degacc.at[idx.at[j]],
                                      ssem).wait()
            return carry
        lax.fori_loop(0, _RET // _KP, chunk, 0)

    plsc.subcore_barrier()

    @pl.when(c == 0)
    def _():
        pltpu.sync_copy(degacc.at[pl.ds(s * _STN, _STN)], stripe_t)

        def newton(i, carry):
            d = stripe_t[i, :]
            bits = plsc.bitcast(d, jnp.int32)
            y = plsc.bitcast(jnp.int32(0x5F3759DF) - (bits >> 1), jnp.float32)
            hd = d * 0.5
            y = y * (1.5 - hd * y * y)
            y = y * (1.5 - hd * y * y)
            y = y * (1.5 - hd * y * y)
            stripe_t[i, :] = y
            return carry
        lax.fori_loop(0, _STN, newton, 0)
        pltpu.sync_copy(stripe_t, dinv_out.at[pl.ds(s * _STN, _STN)])


@jax.jit
def _prep(dst_rs):
    f = pl.kernel(
        _prep_body,
        out_type=jax.ShapeDtypeStruct((_N, 16), jnp.float32),
        mesh=_sc_mesh(),
        compiler_params=pltpu.CompilerParams(use_tc_tiling_on_sc=False, needs_layout_passes=False),
        scratch_types=[
            pltpu.VMEM_SHARED((_N, 16), jnp.float32),
            pltpu.VMEM((_CW, 16), jnp.float32),
            pltpu.VMEM((_KP, _CW), jnp.int32),
            pltpu.VMEM((_STN, 16), jnp.float32),
            pltpu.SemaphoreType.DMA,
        ],
    )
    return f(dst_rs)


def _agg_body(g_lo, g_hi, src_rs, dst_rs, out_lo, out_hi,
              acc, idx_g, idx_s, rows, gsem, ssem, isem):
    c = lax.axis_index("c")
    s = lax.axis_index("s")

    @pl.when(c == 0)
    def _():
        pltpu.sync_copy(g_lo.at[pl.ds(s * _STN, _STN)],
                        acc.at[pl.ds(s * _STN, _STN)])

    @pl.when(c == 1)
    def _():
        pltpu.sync_copy(g_hi.at[pl.ds(s * _STN, _STN)],
                        acc.at[pl.ds(s * _STN, _STN)])
    plsc.subcore_barrier()

    def edge_loop(table):
        base = s * _RET
        nchunk = _RET // _K
        pltpu.async_copy(src_rs.at[pl.ds(base, _K)], idx_g.at[0], isem)
        pltpu.async_copy(dst_rs.at[pl.ds(base, _K)], idx_s.at[0], isem)

        def chunk(cc, carry):
            cur = lax.rem(cc, 2)
            nxt = lax.rem(cc + 1, 2)
            pltpu.make_async_copy(src_rs.at[pl.ds(base, _K)],
                                  idx_g.at[cur], isem).wait()
            pltpu.make_async_copy(dst_rs.at[pl.ds(base, _K)],
                                  idx_s.at[cur], isem).wait()

            @pl.when(cc + 1 < nchunk)
            def _():
                r1 = base + (cc + 1) * _K
                pltpu.async_copy(src_rs.at[pl.ds(r1, _K)], idx_g.at[nxt],
                                 isem)
                pltpu.async_copy(dst_rs.at[pl.ds(r1, _K)], idx_s.at[nxt],
                                 isem)
            for j in range(_K):
                pltpu.async_copy(table.at[idx_g.at[cur, j]], rows.at[j], gsem)
            for j in range(_K):
                pltpu.make_async_copy(table.at[idx_g.at[cur, j]], rows.at[j],
                                      gsem).wait()
                pltpu.async_copy(rows.at[j], acc.at[idx_s.at[cur, j]], ssem,
                                 add=True)
            for j in range(_K):
                pltpu.make_async_copy(rows.at[j], acc.at[idx_s.at[cur, j]],
                                      ssem).wait()
            return carry
        lax.fori_loop(0, nchunk, chunk, 0)

    @pl.when(c == 0)
    def _():
        edge_loop(g_lo)

    @pl.when(c == 1)
    def _():
        edge_loop(g_hi)

    plsc.subcore_barrier()

    @pl.when(c == 0)
    def _():
        pltpu.sync_copy(acc.at[pl.ds(s * _STN, _STN)],
                        out_lo.at[pl.ds(s * _STN, _STN)])

    @pl.when(c == 1)
    def _():
        pltpu.sync_copy(acc.at[pl.ds(s * _STN, _STN)],
                        out_hi.at[pl.ds(s * _STN, _STN)])


@jax.jit
def _agg(g_lo, g_hi, src_rs, dst_rs):
    f = pl.kernel(
        _agg_body,
        out_type=(jax.ShapeDtypeStruct((_N, _HF), jnp.float32),
                  jax.ShapeDtypeStruct((_N, _HF), jnp.float32)),
        mesh=_sc_mesh(),
        compiler_params=pltpu.CompilerParams(use_tc_tiling_on_sc=False, needs_layout_passes=False),
        scratch_types=[
            pltpu.VMEM_SHARED((_N, _HF), jnp.float32),
            pltpu.VMEM((2, _K, _CW), jnp.int32),
            pltpu.VMEM((2, _K, _CW), jnp.int32),
            pltpu.VMEM((_K, _CW, _HF), jnp.float32),
            pltpu.SemaphoreType.DMA,
            pltpu.SemaphoreType.DMA,
            pltpu.SemaphoreType.DMA,
        ],
    )
    return f(g_lo, g_hi, src_rs, dst_rs)


_BLK = 5000


def _tc1_body(x_ref, w_ref, dinv_ref, lo_ref, hi_ref):
    g = jnp.dot(x_ref[...], w_ref[...], preferred_element_type=jnp.float32)
    g = g * dinv_ref[:, :1]
    lo_ref[...] = g[:, :_HF]
    hi_ref[...] = g[:, _HF:]


@jax.jit
def _tc1(x, W1, dinv_w):
    return pl.pallas_call(
        _tc1_body,
        grid=(_N // _BLK,),
        in_specs=[
            pl.BlockSpec((_BLK, _DIN), lambda i: (i, 0)),
            pl.BlockSpec((_DIN, _DH), lambda i: (0, 0)),
            pl.BlockSpec((_BLK, 16), lambda i: (i, 0)),
        ],
        out_specs=[pl.BlockSpec((_BLK, _HF), lambda i: (i, 0)),
                   pl.BlockSpec((_BLK, _HF), lambda i: (i, 0))],
        out_shape=[jax.ShapeDtypeStruct((_N, _HF), jnp.float32),
                   jax.ShapeDtypeStruct((_N, _HF), jnp.float32)],
    )(x, W1, dinv_w)


def _tcmid_body(slo_ref, shi_ref, dinv_ref, b_ref, w_ref,
                lo_ref, hi_ref):
    dinv = dinv_ref[:, :1]
    sf = jnp.concatenate([slo_ref[...], shi_ref[...]], axis=1)
    xn = jnp.maximum(dinv * sf + b_ref[...], 0.0)
    g2 = jnp.dot(xn, w_ref[...], preferred_element_type=jnp.float32) * dinv
    lo_ref[...] = g2[:, :_HF]
    hi_ref[...] = g2[:, _HF:]


@jax.jit
def _tcmid(s_lo, s_hi, dinv_w, b_prev, W):
    blk = lambda i: (i, 0)
    return pl.pallas_call(
        _tcmid_body,
        grid=(_N // _BLK,),
        in_specs=[
            pl.BlockSpec((_BLK, _HF), blk),
            pl.BlockSpec((_BLK, _HF), blk),
            pl.BlockSpec((_BLK, 16), blk),
            pl.BlockSpec((1, _DH), lambda i: (0, 0)),
            pl.BlockSpec((_DH, _DH), lambda i: (0, 0)),
        ],
        out_specs=[pl.BlockSpec((_BLK, _HF), blk),
                   pl.BlockSpec((_BLK, _HF), blk)],
        out_shape=[jax.ShapeDtypeStruct((_N, _HF), jnp.float32),
                   jax.ShapeDtypeStruct((_N, _HF), jnp.float32)],
    )(s_lo, s_hi, dinv_w, b_prev, W)


def _poolmm_body(slo_ref, shi_ref, dinv_ref, batch_ref,
                 b3_ref, wl_ref, bl_ref, out_ref, sum_ref, cnt_ref):
    i = pl.program_id(0)
    dinv = dinv_ref[:, :1]
    sf = jnp.concatenate([slo_ref[...], shi_ref[...]], axis=1)
    h = dinv * sf
    ids = batch_ref[:, :1]
    gidx = lax.broadcasted_iota(jnp.int32, (_BLK, _NG), 1)
    oh = (ids == gidx).astype(jnp.float32)
    psum = lax.dot_general(oh, h, (((0,), (0,)), ((), ())),
                           preferred_element_type=jnp.float32)
    pcnt = jnp.sum(oh, axis=0)[None, :]

    @pl.when(i == 0)
    def _():
        sum_ref[...] = jnp.zeros_like(sum_ref)
        cnt_ref[...] = jnp.zeros_like(cnt_ref)
    sum_ref[...] += psum
    cnt_ref[...] += pcnt

    @pl.when(i == _N // _BLK - 1)
    def _():
        t = sum_ref[...]
        cnt = jnp.reshape(cnt_ref[0], (_NG, 1))
        pooled = (t + cnt * b3_ref[...]) / jnp.maximum(cnt, 1.0)
        out_ref[...] = (jnp.dot(pooled, wl_ref[...],
                                preferred_element_type=jnp.float32)
                        + bl_ref[...])


@jax.jit
def _poolmm(s_lo, s_hi, dinv_w, batch_col, b3, Wl, bl):
    return pl.pallas_call(
        _poolmm_body,
        grid=(_N // _BLK,),
        in_specs=[
            pl.BlockSpec((_BLK, _HF), lambda i: (i, 0)),
            pl.BlockSpec((_BLK, _HF), lambda i: (i, 0)),
            pl.BlockSpec((_BLK, 16), lambda i: (i, 0)),
            pl.BlockSpec((_BLK, 1), lambda i: (i, 0)),
            pl.BlockSpec((1, _DH), lambda i: (0, 0)),
            pl.BlockSpec((_DH, 8), lambda i: (0, 0)),
            pl.BlockSpec((1, 8), lambda i: (0, 0)),
        ],
        out_specs=pl.BlockSpec((_NG, 8), lambda i: (0, 0)),
        out_shape=jax.ShapeDtypeStruct((_NG, 8), jnp.float32),
        scratch_shapes=[
            pltpu.VMEM((_NG, _DH), jnp.float32),
            pltpu.VMEM((1, _NG), jnp.float32),
        ],
    )(s_lo, s_hi, dinv_w, batch_col, b3, Wl, bl)


def kernel(x, edge_index, batch, W1, b1, W2, b2, W3, b3, Wl, bl):
    src = edge_index[0].astype(jnp.int32)
    dst = edge_index[1].astype(jnp.int32)
    src_rs = src.reshape(_RE, _CW)
    dst_rs = dst.reshape(_RE, _CW)
    batch_col = batch.astype(jnp.int32).reshape(_N, 1)

    dinv_w = _prep(dst_rs)
    g1_lo, g1_hi = _tc1(x, W1, dinv_w)
    s1_lo, s1_hi = _agg(g1_lo, g1_hi, src_rs, dst_rs)
    g2_lo, g2_hi = _tcmid(s1_lo, s1_hi, dinv_w, b1.reshape(1, _DH), W2)
    s2_lo, s2_hi = _agg(g2_lo, g2_hi, src_rs, dst_rs)
    g3_lo, g3_hi = _tcmid(s2_lo, s2_hi, dinv_w, b2.reshape(1, _DH), W3)
    s3_lo, s3_hi = _agg(g3_lo, g3_hi, src_rs, dst_rs)
    return _poolmm(s3_lo, s3_hi, dinv_w, batch_col,
                   b3.reshape(1, _DH), Wl, bl.reshape(1, 8))
